# baseline (speedup 1.0000x reference)
import functools

import jax
import jax.numpy as jnp
from jax import lax
from jax.experimental import pallas as pl
from jax.experimental.pallas import tpu as pltpu
from jax.experimental.pallas import tpu_sc as plsc

N = 10000
E = 320000
D = 128
R = 8
B = 8

NC = 2
NS = 16
NW = NC * NS
LANES = 16

CH = 80
EPW = E // NW
NCH = EPW // CH
RPT = 624


BN = 1000
NB = N // BN


def _wcomb_body(coeff_ref, bases_ref, w_ref):
    w_ref[...] = jnp.dot(coeff_ref[...], bases_ref[...],
                         preferred_element_type=jnp.float32)


def _combine_w(coeff, bases):
    w2 = pl.pallas_call(
        _wcomb_body,
        out_shape=jax.ShapeDtypeStruct((R, D * D), jnp.float32),
    )(coeff, bases.reshape(B, D * D))
    return w2.reshape(R, D, D)


def _mm0_body(h_ref, w_ref, out_ref):
    out_ref[0] = jnp.dot(h_ref[...], w_ref[0],
                         preferred_element_type=jnp.float32)


def _all_transform0(h, w):
    return pl.pallas_call(
        _mm0_body,
        grid=(NB, R),
        in_specs=[
            pl.BlockSpec((BN, D), lambda nb, r: (nb, 0)),
            pl.BlockSpec((1, D, D), lambda nb, r: (r, 0, 0)),
        ],
        out_specs=pl.BlockSpec((1, BN, D), lambda nb, r: (r, nb, 0)),
        out_shape=jax.ShapeDtypeStruct((R, N, D), jnp.float32),
    )(h, w)


def _mm1_body(p_ref, b_ref, w_ref, out_ref, h_s):
    r = pl.program_id(1)

    @pl.when(r == 0)
    def _build_h():
        h_s[...] = jnp.maximum(p_ref[0] + p_ref[1] + b_ref[...], 0.0)

    out_ref[0] = jnp.dot(h_s[...], w_ref[0],
                         preferred_element_type=jnp.float32)


def _all_transform1(p, bias, w):
    return pl.pallas_call(
        _mm1_body,
        grid=(NB, R),
        in_specs=[
            pl.BlockSpec((NC, BN, D), lambda nb, r: (0, nb, 0)),
            pl.BlockSpec((1, D), lambda nb, r: (0, 0)),
            pl.BlockSpec((1, D, D), lambda nb, r: (r, 0, 0)),
        ],
        out_specs=pl.BlockSpec((1, BN, D), lambda nb, r: (r, nb, 0)),
        out_shape=jax.ShapeDtypeStruct((R, N, D), jnp.float32),
        scratch_shapes=[pltpu.VMEM((BN, D), jnp.float32)],
    )(p, bias.reshape(1, D), w)


def _prep_body(src_ref, et_ref, dst_ref, rid_ref, dstf_ref):
    rid_ref[...] = et_ref[...] * N + src_ref[...]
    dstf_ref[...] = dst_ref[...].astype(jnp.float32)


def _prep_edges(edge_index, etype):
    return pl.pallas_call(
        _prep_body,
        out_shape=(jax.ShapeDtypeStruct((1, E), jnp.int32),
                   jax.ShapeDtypeStruct((1, E), jnp.float32)),
    )(edge_index[0].reshape(1, E), etype.reshape(1, E),
      edge_index[1].reshape(1, E))


def _comb_body(p_ref, b_ref, o_ref):
    o_ref[...] = p_ref[0] + p_ref[1] + b_ref[...]


def _combine_partials(p, bias):
    return pl.pallas_call(
        _comb_body,
        grid=(NB,),
        in_specs=[
            pl.BlockSpec((NC, BN, D), lambda nb: (0, nb, 0)),
            pl.BlockSpec((1, D), lambda nb: (0, 0)),
        ],
        out_specs=pl.BlockSpec((BN, D), lambda nb: (nb, 0)),
        out_shape=jax.ShapeDtypeStruct((N, D), jnp.float32),
    )(p, bias.reshape(1, D))


def _edge_scatter_body(rid_hbm, fd_hbm, table_hbm, out_hbm,
                       rid_v, fd_v, dst_ix, rows_v, acc,
                       sem_m, sem_g, sem_s):
    cid = lax.axis_index("c")
    sid = lax.axis_index("s")

    def _zb(i, _):
        for c in range(D // LANES):
            rows_v[0, i, pl.ds(c * LANES, LANES)] = jnp.zeros((LANES,),
                                                              jnp.float32)
        return 0
    lax.fori_loop(0, CH, _zb, 0)
    row0 = sid * RPT
    for j in range(RPT // CH):
        pltpu.sync_copy(rows_v.at[0], acc.at[pl.ds(row0 + j * CH, CH)])
    rem = RPT - (RPT // CH) * CH
    pltpu.sync_copy(rows_v.at[0].at[pl.ds(0, rem)],
                    acc.at[pl.ds(row0 + RPT - rem, rem)])

    @pl.when(sid == 0)
    def _zero_tail():
        pltpu.sync_copy(rows_v.at[0].at[pl.ds(0, N - NS * RPT)],
                        acc.at[pl.ds(NS * RPT, N - NS * RPT)])
    plsc.subcore_barrier()

    wid = cid * NS + sid

    pltpu.sync_copy(rid_hbm.at[pl.ds(wid * EPW, EPW)], rid_v)


    def _meta_start(i, a):
        pltpu.async_copy(fd_hbm.at[wid, i], fd_v.at[a], sem_m)

    def _meta_wait(i, a):
        pltpu.make_async_copy(fd_hbm.at[wid, i], fd_v.at[a], sem_m).wait()

    def _build_chunk(a):
        for g in range(CH // LANES):
            sl = pl.ds(g * LANES, LANES)
            dst_ix[a, sl] = fd_v[a, 0, sl].astype(jnp.int32)

    def _gather_desc(i, a):
        return pltpu.make_async_copy(table_hbm.at[rid_v.at[pl.ds(i * CH,
                                                                 CH)]],
                                     rows_v.at[a], sem_g)

    def _scale(b):
        for g in range(CH // LANES):
            nv = fd_v[b, 1, pl.ds(g * LANES, LANES)]
            for j in range(LANES):
                ns = nv[j]
                k = g * LANES + j
                for c in range(D // LANES):
                    sl = pl.ds(c * LANES, LANES)
                    rows_v[b, k, sl] = rows_v[b, k, sl] * ns

    def _scatter_desc(b):
        return pltpu.make_async_copy(rows_v.at[b], acc.at[dst_ix.at[b]],
                                     sem_s)

    _meta_start(0, 0)
    _meta_wait(0, 0)
    _build_chunk(0)
    _gather_desc(0, 0).start()
    _meta_start(1, 1)

    def _cb(i, _):
        a = lax.rem(i, 3)
        b = lax.rem(i + 2, 3)

        @pl.when(i >= 3)
        def _drain():
            _scatter_desc(a).wait()
        _meta_wait(i, a)
        _build_chunk(a)
        _gather_desc(i, a).start()

        @pl.when(i < NCH - 1)
        def _prefetch():
            _meta_start(i + 1, lax.rem(i + 1, 3))

        _gather_desc(i - 1, b).wait()
        _scale(b)
        _scatter_desc(b).start(add=True)
        return 0
    lax.fori_loop(1, NCH, _cb, 0)

    last = (NCH - 1) % 3
    _gather_desc(NCH - 1, last).wait()
    _scale(last)
    _scatter_desc(last).start(add=True)
    for c in (NCH - 3, NCH - 2, NCH - 1):
        _scatter_desc(c % 3).wait()

    plsc.subcore_barrier()
    sl0 = pl.ds(row0, RPT)
    pltpu.sync_copy(acc.at[sl0], out_hbm.at[cid].at[sl0])

    @pl.when(sid == 0)
    def _pub_tail():
        tl = pl.ds(NS * RPT, N - NS * RPT)
        pltpu.sync_copy(acc.at[tl], out_hbm.at[cid].at[tl])


_edge_scatter = functools.partial(
    pl.kernel,
    out_type=jax.ShapeDtypeStruct((NC, N, D), jnp.float32),
    mesh=plsc.VectorSubcoreMesh(core_axis_name="c", subcore_axis_name="s"),
    scratch_types=[
        pltpu.VMEM((EPW,), jnp.int32),
        pltpu.VMEM((3, 2, CH), jnp.float32),
        pltpu.VMEM((3, CH), jnp.int32),
        pltpu.VMEM((3, CH, D), jnp.float32),
        pltpu.VMEM_SHARED((N, D), jnp.float32),
        pltpu.SemaphoreType.DMA,
        pltpu.SemaphoreType.DMA,
        pltpu.SemaphoreType.DMA,
    ],
)(_edge_scatter_body)


def kernel(feat, edge_index, etype, norm, coeff0, bases0, bias0,
           coeff1, bases1, bias1):
    rid, dstf = _prep_edges(edge_index, etype)
    rid = rid.reshape(E)
    fd = (jnp.stack([dstf.reshape(E), norm.reshape(E)], 0)
          .reshape(2, NW, NCH, CH).transpose(1, 2, 0, 3))

    w0 = _combine_w(coeff0, bases0)
    w1 = _combine_w(coeff1, bases1)
    t0 = _all_transform0(feat, w0).reshape(R * N, D)
    p0 = _edge_scatter(rid, fd, t0)
    t1 = _all_transform1(p0, bias0, w1).reshape(R * N, D)
    p1 = _edge_scatter(rid, fd, t1)
    return _combine_partials(p1, bias1)


# speedup vs baseline: 1.0032x; 1.0032x over previous
import functools

import jax
import jax.numpy as jnp
from jax import lax
from jax.experimental import pallas as pl
from jax.experimental.pallas import tpu as pltpu
from jax.experimental.pallas import tpu_sc as plsc

N = 10000
E = 320000
D = 128
R = 8
B = 8

NC = 2
NS = 16
NW = NC * NS
LANES = 16

CH = 80
EPW = E // NW
NCH = EPW // CH
RPT = 624


BN = 1000
NB = N // BN


def _setup_body(src_ref, et_ref, dst_ref, c0_ref, b0_ref, c1_ref, b1_ref,
                rid_ref, dstf_ref, w0_ref, w1_ref):
    rid_ref[...] = et_ref[...] * N + src_ref[...]
    dstf_ref[...] = dst_ref[...].astype(jnp.float32)
    w0_ref[...] = jnp.dot(c0_ref[...], b0_ref[...],
                          preferred_element_type=jnp.float32)
    w1_ref[...] = jnp.dot(c1_ref[...], b1_ref[...],
                          preferred_element_type=jnp.float32)


def _setup(edge_index, etype, coeff0, bases0, coeff1, bases1):
    rid, dstf, w0, w1 = pl.pallas_call(
        _setup_body,
        out_shape=(jax.ShapeDtypeStruct((1, E), jnp.int32),
                   jax.ShapeDtypeStruct((1, E), jnp.float32),
                   jax.ShapeDtypeStruct((R, D * D), jnp.float32),
                   jax.ShapeDtypeStruct((R, D * D), jnp.float32)),
    )(edge_index[0].reshape(1, E), etype.reshape(1, E),
      edge_index[1].reshape(1, E), coeff0, bases0.reshape(B, D * D),
      coeff1, bases1.reshape(B, D * D))
    return rid, dstf, w0.reshape(R, D, D), w1.reshape(R, D, D)


def _mm0_body(h_ref, w_ref, out_ref):
    out_ref[0] = jnp.dot(h_ref[...], w_ref[0],
                         preferred_element_type=jnp.float32)


def _all_transform0(h, w):
    return pl.pallas_call(
        _mm0_body,
        grid=(NB, R),
        in_specs=[
            pl.BlockSpec((BN, D), lambda nb, r: (nb, 0)),
            pl.BlockSpec((1, D, D), lambda nb, r: (r, 0, 0)),
        ],
        out_specs=pl.BlockSpec((1, BN, D), lambda nb, r: (r, nb, 0)),
        out_shape=jax.ShapeDtypeStruct((R, N, D), jnp.float32),
    )(h, w)


def _mm1_body(p_ref, b_ref, w_ref, out_ref, h_s):
    r = pl.program_id(1)

    @pl.when(r == 0)
    def _build_h():
        h_s[...] = jnp.maximum(p_ref[0] + p_ref[1] + b_ref[...], 0.0)

    out_ref[0] = jnp.dot(h_s[...], w_ref[0],
                         preferred_element_type=jnp.float32)


def _all_transform1(p, bias, w):
    return pl.pallas_call(
        _mm1_body,
        grid=(NB, R),
        in_specs=[
            pl.BlockSpec((NC, BN, D), lambda nb, r: (0, nb, 0)),
            pl.BlockSpec((1, D), lambda nb, r: (0, 0)),
            pl.BlockSpec((1, D, D), lambda nb, r: (r, 0, 0)),
        ],
        out_specs=pl.BlockSpec((1, BN, D), lambda nb, r: (r, nb, 0)),
        out_shape=jax.ShapeDtypeStruct((R, N, D), jnp.float32),
        scratch_shapes=[pltpu.VMEM((BN, D), jnp.float32)],
    )(p, bias.reshape(1, D), w)


def _comb_body(p_ref, b_ref, o_ref):
    o_ref[...] = p_ref[0] + p_ref[1] + b_ref[...]


def _combine_partials(p, bias):
    return pl.pallas_call(
        _comb_body,
        grid=(NB,),
        in_specs=[
            pl.BlockSpec((NC, BN, D), lambda nb: (0, nb, 0)),
            pl.BlockSpec((1, D), lambda nb: (0, 0)),
        ],
        out_specs=pl.BlockSpec((BN, D), lambda nb: (nb, 0)),
        out_shape=jax.ShapeDtypeStruct((N, D), jnp.float32),
    )(p, bias.reshape(1, D))


def _edge_scatter_body(rid_hbm, fd_hbm, table_hbm, out_hbm,
                       rid_v, fd_v, dst_ix, rows_v, acc,
                       sem_m, sem_g, sem_s):
    cid = lax.axis_index("c")
    sid = lax.axis_index("s")

    def _zb(i, _):
        for c in range(D // LANES):
            rows_v[0, i, pl.ds(c * LANES, LANES)] = jnp.zeros((LANES,),
                                                              jnp.float32)
        return 0
    lax.fori_loop(0, CH, _zb, 0)
    row0 = sid * RPT
    for j in range(RPT // CH):
        pltpu.sync_copy(rows_v.at[0], acc.at[pl.ds(row0 + j * CH, CH)])
    rem = RPT - (RPT // CH) * CH
    pltpu.sync_copy(rows_v.at[0].at[pl.ds(0, rem)],
                    acc.at[pl.ds(row0 + RPT - rem, rem)])

    @pl.when(sid == 0)
    def _zero_tail():
        pltpu.sync_copy(rows_v.at[0].at[pl.ds(0, N - NS * RPT)],
                        acc.at[pl.ds(NS * RPT, N - NS * RPT)])
    plsc.subcore_barrier()

    wid = cid * NS + sid

    pltpu.sync_copy(rid_hbm.at[pl.ds(wid * EPW, EPW)], rid_v)


    def _meta_start(i, a):
        pltpu.async_copy(fd_hbm.at[wid, i], fd_v.at[a], sem_m)

    def _meta_wait(i, a):
        pltpu.make_async_copy(fd_hbm.at[wid, i], fd_v.at[a], sem_m).wait()

    def _build_chunk(a):
        for g in range(CH // LANES):
            sl = pl.ds(g * LANES, LANES)
            dst_ix[a, sl] = fd_v[a, 0, sl].astype(jnp.int32)

    def _gather_desc(i, a):
        return pltpu.make_async_copy(table_hbm.at[rid_v.at[pl.ds(i * CH,
                                                                 CH)]],
                                     rows_v.at[a], sem_g)

    def _scale(b):
        for g in range(CH // LANES):
            nv = fd_v[b, 1, pl.ds(g * LANES, LANES)]
            for j in range(LANES):
                ns = nv[j]
                k = g * LANES + j
                for c in range(D // LANES):
                    sl = pl.ds(c * LANES, LANES)
                    rows_v[b, k, sl] = rows_v[b, k, sl] * ns

    def _scatter_desc(b):
        return pltpu.make_async_copy(rows_v.at[b], acc.at[dst_ix.at[b]],
                                     sem_s)

    _meta_start(0, 0)
    _meta_wait(0, 0)
    _build_chunk(0)
    _gather_desc(0, 0).start()
    _meta_start(1, 1)

    def _cb(i, _):
        a = lax.rem(i, 3)
        b = lax.rem(i + 2, 3)

        @pl.when(i >= 3)
        def _drain():
            _scatter_desc(a).wait()
        _meta_wait(i, a)
        _build_chunk(a)
        _gather_desc(i, a).start()

        @pl.when(i < NCH - 1)
        def _prefetch():
            _meta_start(i + 1, lax.rem(i + 1, 3))

        _gather_desc(i - 1, b).wait()
        _scale(b)
        _scatter_desc(b).start(add=True)
        return 0
    lax.fori_loop(1, NCH, _cb, 0)

    last = (NCH - 1) % 3
    _gather_desc(NCH - 1, last).wait()
    _scale(last)
    _scatter_desc(last).start(add=True)
    for c in (NCH - 3, NCH - 2, NCH - 1):
        _scatter_desc(c % 3).wait()

    plsc.subcore_barrier()
    sl0 = pl.ds(row0, RPT)
    pltpu.sync_copy(acc.at[sl0], out_hbm.at[cid].at[sl0])

    @pl.when(sid == 0)
    def _pub_tail():
        tl = pl.ds(NS * RPT, N - NS * RPT)
        pltpu.sync_copy(acc.at[tl], out_hbm.at[cid].at[tl])


_edge_scatter = functools.partial(
    pl.kernel,
    out_type=jax.ShapeDtypeStruct((NC, N, D), jnp.float32),
    mesh=plsc.VectorSubcoreMesh(core_axis_name="c", subcore_axis_name="s"),
    scratch_types=[
        pltpu.VMEM((EPW,), jnp.int32),
        pltpu.VMEM((3, 2, CH), jnp.float32),
        pltpu.VMEM((3, CH), jnp.int32),
        pltpu.VMEM((3, CH, D), jnp.float32),
        pltpu.VMEM_SHARED((N, D), jnp.float32),
        pltpu.SemaphoreType.DMA,
        pltpu.SemaphoreType.DMA,
        pltpu.SemaphoreType.DMA,
    ],
)(_edge_scatter_body)


def kernel(feat, edge_index, etype, norm, coeff0, bases0, bias0,
           coeff1, bases1, bias1):
    rid, dstf, w0, w1 = _setup(edge_index, etype, coeff0, bases0,
                               coeff1, bases1)
    rid = rid.reshape(E)
    fd = (jnp.stack([dstf.reshape(E), norm.reshape(E)], 0)
          .reshape(2, NW, NCH, CH).transpose(1, 2, 0, 3))

    t0 = _all_transform0(feat, w0).reshape(R * N, D)
    p0 = _edge_scatter(rid, fd, t0)
    t1 = _all_transform1(p0, bias0, w1).reshape(R * N, D)
    p1 = _edge_scatter(rid, fd, t1)
    return _combine_partials(p1, bias1)


# speedup vs baseline: 1.0695x; 1.0661x over previous
import functools

import jax
import jax.numpy as jnp
from jax import lax
from jax.experimental import pallas as pl
from jax.experimental.pallas import tpu as pltpu
from jax.experimental.pallas import tpu_sc as plsc

N = 10000
E = 320000
D = 128
R = 8
B = 8

NC = 2
NS = 16
NW = NC * NS
LANES = 16

CH = 80
EPW = E // NW
NCH = EPW // CH
RPT = 624


BN = 1000
NB = N // BN


def _wcomb_body(coeff_ref, bases_ref, w_ref):
    w_ref[...] = jnp.dot(coeff_ref[...], bases_ref[...],
                         preferred_element_type=jnp.float32)


def _combine_w(coeff, bases):
    w2 = pl.pallas_call(
        _wcomb_body,
        out_shape=jax.ShapeDtypeStruct((R, D * D), jnp.float32),
    )(coeff, bases.reshape(B, D * D))
    return w2.reshape(R, D, D)


def _mm0_body(h_ref, w_ref, out_ref):
    out_ref[0] = jnp.dot(h_ref[...], w_ref[0],
                         preferred_element_type=jnp.float32)


def _all_transform0(h, w):
    return pl.pallas_call(
        _mm0_body,
        grid=(NB, R),
        in_specs=[
            pl.BlockSpec((BN, D), lambda nb, r: (nb, 0)),
            pl.BlockSpec((1, D, D), lambda nb, r: (r, 0, 0)),
        ],
        out_specs=pl.BlockSpec((1, BN, D), lambda nb, r: (r, nb, 0)),
        out_shape=jax.ShapeDtypeStruct((R, N, D), jnp.float32),
    )(h, w)


def _mm1_body(p_ref, b_ref, w_ref, out_ref, h_s):
    r = pl.program_id(1)

    @pl.when(r == 0)
    def _build_h():
        h_s[...] = jnp.maximum(p_ref[0] + p_ref[1] + b_ref[...], 0.0)

    out_ref[0] = jnp.dot(h_s[...], w_ref[0],
                         preferred_element_type=jnp.float32)


def _all_transform1(p, bias, w):
    return pl.pallas_call(
        _mm1_body,
        grid=(NB, R),
        in_specs=[
            pl.BlockSpec((NC, BN, D), lambda nb, r: (0, nb, 0)),
            pl.BlockSpec((1, D), lambda nb, r: (0, 0)),
            pl.BlockSpec((1, D, D), lambda nb, r: (r, 0, 0)),
        ],
        out_specs=pl.BlockSpec((1, BN, D), lambda nb, r: (r, nb, 0)),
        out_shape=jax.ShapeDtypeStruct((R, N, D), jnp.float32),
        scratch_shapes=[pltpu.VMEM((BN, D), jnp.float32)],
    )(p, bias.reshape(1, D), w)


def _comb_body(p_ref, b_ref, o_ref):
    o_ref[...] = p_ref[0] + p_ref[1] + b_ref[...]


def _combine_partials(p, bias):
    return pl.pallas_call(
        _comb_body,
        grid=(NB,),
        in_specs=[
            pl.BlockSpec((NC, BN, D), lambda nb: (0, nb, 0)),
            pl.BlockSpec((1, D), lambda nb: (0, 0)),
        ],
        out_specs=pl.BlockSpec((BN, D), lambda nb: (nb, 0)),
        out_shape=jax.ShapeDtypeStruct((N, D), jnp.float32),
    )(p, bias.reshape(1, D))


def _edge_scatter_body(edata_hbm, norm_hbm, table_hbm, out_hbm,
                       ed_v, rid_v, dst_ix, norm_sm, rows_v, acc,
                       sem_m, sem_g, sem_s):
    cid = lax.axis_index("c")
    sid = lax.axis_index("s")

    def _zb(i, _):
        for c in range(D // LANES):
            rows_v[0, i, pl.ds(c * LANES, LANES)] = jnp.zeros((LANES,),
                                                              jnp.float32)
        return 0
    lax.fori_loop(0, CH, _zb, 0)
    row0 = sid * RPT
    for j in range(RPT // CH):
        pltpu.sync_copy(rows_v.at[0], acc.at[pl.ds(row0 + j * CH, CH)])
    rem = RPT - (RPT // CH) * CH
    pltpu.sync_copy(rows_v.at[0].at[pl.ds(0, rem)],
                    acc.at[pl.ds(row0 + RPT - rem, rem)])

    @pl.when(sid == 0)
    def _zero_tail():
        pltpu.sync_copy(rows_v.at[0].at[pl.ds(0, N - NS * RPT)],
                        acc.at[pl.ds(NS * RPT, N - NS * RPT)])
    plsc.subcore_barrier()

    wid = cid * NS + sid


    def _meta_start(i, e, m):
        pltpu.async_copy(edata_hbm.at[wid, i], ed_v.at[e], sem_m)
        pltpu.async_copy(norm_hbm.at[wid, i], norm_sm.at[m], sem_m)

    def _meta_wait(i, e, m):
        pltpu.make_async_copy(edata_hbm.at[wid, i], ed_v.at[e], sem_m).wait()
        pltpu.make_async_copy(norm_hbm.at[wid, i], norm_sm.at[m],
                              sem_m).wait()

    def _build_chunk(e, m):
        for g in range(CH // LANES):
            sl = pl.ds(g * LANES, LANES)
            rid_v[m, sl] = ed_v[e, 1, sl] * N + ed_v[e, 0, sl]
            dst_ix[m, sl] = ed_v[e, 2, sl]

    def _gather_desc(m, q):
        return pltpu.make_async_copy(table_hbm.at[rid_v.at[m]],
                                     rows_v.at[q], sem_g)

    def _scale(q, m):
        for g in range(CH // LANES):
            nv = norm_sm[m, pl.ds(g * LANES, LANES)]
            for j in range(LANES):
                ns = nv[j]
                k = g * LANES + j
                for c in range(D // LANES):
                    sl = pl.ds(c * LANES, LANES)
                    rows_v[q, k, sl] = rows_v[q, k, sl] * ns

    def _scatter_desc(q, m):
        return pltpu.make_async_copy(rows_v.at[q], acc.at[dst_ix.at[m]],
                                     sem_s)

    _meta_start(0, 0, 0)
    _meta_start(1, 1, 1)
    _meta_wait(0, 0, 0)
    _build_chunk(0, 0)
    _gather_desc(0, 0).start()
    _meta_start(2, 2, 2)

    def _cb(i, _):
        e = lax.rem(i, 4)
        m = lax.rem(i, 8)
        q = lax.rem(i, 4)
        mp = lax.rem(i + 6, 8)
        qp = lax.rem(i + 2, 4)

        @pl.when(i >= 4)
        def _drain():
            _scatter_desc(q, lax.rem(i + 4, 8)).wait()
        _meta_wait(i, e, m)
        _build_chunk(e, m)
        _gather_desc(m, q).start()

        @pl.when(i < NCH - 2)
        def _prefetch():
            _meta_start(i + 2, lax.rem(i + 2, 4), lax.rem(i + 2, 8))

        @pl.when(i >= 2)
        def _process():
            _gather_desc(mp, qp).wait()
            _scale(qp, mp)
            _scatter_desc(qp, mp).start(add=True)
        return 0
    lax.fori_loop(1, NCH, _cb, 0)

    for c in (NCH - 2, NCH - 1):
        _gather_desc(c % 8, c % 4).wait()
        _scale(c % 4, c % 8)
        _scatter_desc(c % 4, c % 8).start(add=True)
    for c in (NCH - 4, NCH - 3, NCH - 2, NCH - 1):
        _scatter_desc(c % 4, c % 8).wait()

    plsc.subcore_barrier()
    sl0 = pl.ds(row0, RPT)
    pltpu.sync_copy(acc.at[sl0], out_hbm.at[cid].at[sl0])

    @pl.when(sid == 0)
    def _pub_tail():
        tl = pl.ds(NS * RPT, N - NS * RPT)
        pltpu.sync_copy(acc.at[tl], out_hbm.at[cid].at[tl])


_edge_scatter = functools.partial(
    pl.kernel,
    out_type=jax.ShapeDtypeStruct((NC, N, D), jnp.float32),
    mesh=plsc.VectorSubcoreMesh(core_axis_name="c", subcore_axis_name="s"),
    scratch_types=[
        pltpu.VMEM((4, 3, CH), jnp.int32),
        pltpu.VMEM((8, CH), jnp.int32),
        pltpu.VMEM((8, CH), jnp.int32),
        pltpu.VMEM((8, CH), jnp.float32),
        pltpu.VMEM((4, CH, D), jnp.float32),
        pltpu.VMEM_SHARED((N, D), jnp.float32),
        pltpu.SemaphoreType.DMA,
        pltpu.SemaphoreType.DMA,
        pltpu.SemaphoreType.DMA,
    ],
)(_edge_scatter_body)


def kernel(feat, edge_index, etype, norm, coeff0, bases0, bias0,
           coeff1, bases1, bias1):
    edata = (jnp.stack([edge_index[0], etype, edge_index[1]], 0)
             .reshape(3, NW, NCH, CH).transpose(1, 2, 0, 3))
    norm3 = norm.reshape(NW, NCH, CH)

    w0 = _combine_w(coeff0, bases0)
    w1 = _combine_w(coeff1, bases1)
    t0 = _all_transform0(feat, w0).reshape(R * N, D)
    p0 = _edge_scatter(edata, norm3, t0)
    t1 = _all_transform1(p0, bias0, w1).reshape(R * N, D)
    p1 = _edge_scatter(edata, norm3, t1)
    return _combine_partials(p1, bias1)


# speedup vs baseline: 1.0719x; 1.0022x over previous
import functools

import jax
import jax.numpy as jnp
from jax import lax
from jax.experimental import pallas as pl
from jax.experimental.pallas import tpu as pltpu
from jax.experimental.pallas import tpu_sc as plsc

N = 10000
E = 320000
D = 128
R = 8
B = 8

NC = 2
NS = 16
NW = NC * NS
LANES = 16

CH = 80
EPW = E // NW
NCH = EPW // CH
RPT = 624


BN = 1000
NB = N // BN


def _wcomb_body(coeff_ref, bases_ref, w_ref):
    w_ref[...] = jnp.dot(coeff_ref[...], bases_ref[...],
                         preferred_element_type=jnp.float32)


def _combine_w(coeff, bases):
    w2 = pl.pallas_call(
        _wcomb_body,
        out_shape=jax.ShapeDtypeStruct((R, D * D), jnp.float32),
    )(coeff, bases.reshape(B, D * D))
    return w2.reshape(R, D, D)


def _mm0_body(h_ref, w_ref, out_ref):
    out_ref[0] = jnp.dot(h_ref[...], w_ref[0],
                         preferred_element_type=jnp.float32)


def _all_transform0(h, w):
    return pl.pallas_call(
        _mm0_body,
        grid=(NB, R),
        in_specs=[
            pl.BlockSpec((BN, D), lambda nb, r: (nb, 0)),
            pl.BlockSpec((1, D, D), lambda nb, r: (r, 0, 0)),
        ],
        out_specs=pl.BlockSpec((1, BN, D), lambda nb, r: (r, nb, 0)),
        out_shape=jax.ShapeDtypeStruct((R, N, D), jnp.float32),
    )(h, w)


def _mm1_body(p_ref, b_ref, w_ref, out_ref, h_s):
    r = pl.program_id(1)

    @pl.when(r == 0)
    def _build_h():
        h_s[...] = jnp.maximum(p_ref[0] + p_ref[1] + b_ref[...], 0.0)

    out_ref[0] = jnp.dot(h_s[...], w_ref[0],
                         preferred_element_type=jnp.float32)


def _all_transform1(p, bias, w):
    return pl.pallas_call(
        _mm1_body,
        grid=(NB, R),
        in_specs=[
            pl.BlockSpec((NC, BN, D), lambda nb, r: (0, nb, 0)),
            pl.BlockSpec((1, D), lambda nb, r: (0, 0)),
            pl.BlockSpec((1, D, D), lambda nb, r: (r, 0, 0)),
        ],
        out_specs=pl.BlockSpec((1, BN, D), lambda nb, r: (r, nb, 0)),
        out_shape=jax.ShapeDtypeStruct((R, N, D), jnp.float32),
        scratch_shapes=[pltpu.VMEM((BN, D), jnp.float32)],
    )(p, bias.reshape(1, D), w)


def _comb_body(p_ref, b_ref, o_ref):
    o_ref[...] = p_ref[0] + p_ref[1] + b_ref[...]


def _combine_partials(p, bias):
    return pl.pallas_call(
        _comb_body,
        grid=(NB,),
        in_specs=[
            pl.BlockSpec((NC, BN, D), lambda nb: (0, nb, 0)),
            pl.BlockSpec((1, D), lambda nb: (0, 0)),
        ],
        out_specs=pl.BlockSpec((BN, D), lambda nb: (nb, 0)),
        out_shape=jax.ShapeDtypeStruct((N, D), jnp.float32),
    )(p, bias.reshape(1, D))


def _edge_scatter_body(edata_hbm, norm_hbm, table_hbm, out_hbm,
                       ed_v, rid_v, dst_ix, norm_sm, rows_v, acc,
                       sem_m, sem_g, sem_s):
    cid = lax.axis_index("c")
    sid = lax.axis_index("s")

    wid = cid * NS + sid


    def _meta_start(i, e, m):
        pltpu.async_copy(edata_hbm.at[wid, i], ed_v.at[e], sem_m)
        pltpu.async_copy(norm_hbm.at[wid, i], norm_sm.at[m], sem_m)

    def _meta_wait(i, e, m):
        pltpu.make_async_copy(edata_hbm.at[wid, i], ed_v.at[e], sem_m).wait()
        pltpu.make_async_copy(norm_hbm.at[wid, i], norm_sm.at[m],
                              sem_m).wait()

    def _build_chunk(e, m):
        for g in range(CH // LANES):
            sl = pl.ds(g * LANES, LANES)
            rid_v[m, sl] = ed_v[e, 1, sl] * N + ed_v[e, 0, sl]
            dst_ix[m, sl] = ed_v[e, 2, sl]

    def _gather_desc(m, q):
        return pltpu.make_async_copy(table_hbm.at[rid_v.at[m]],
                                     rows_v.at[q], sem_g)

    def _scale(q, m):
        for g in range(CH // LANES):
            nv = norm_sm[m, pl.ds(g * LANES, LANES)]
            for j in range(LANES):
                ns = nv[j]
                k = g * LANES + j
                for c in range(D // LANES):
                    sl = pl.ds(c * LANES, LANES)
                    rows_v[q, k, sl] = rows_v[q, k, sl] * ns

    def _scatter_desc(q, m):
        return pltpu.make_async_copy(rows_v.at[q], acc.at[dst_ix.at[m]],
                                     sem_s)

    _meta_start(0, 0, 0)
    _meta_start(1, 1, 1)
    _meta_start(2, 2, 2)

    def _zb(i, _):
        for c in range(D // LANES):
            rows_v[0, i, pl.ds(c * LANES, LANES)] = jnp.zeros((LANES,),
                                                              jnp.float32)
        return 0
    lax.fori_loop(0, CH, _zb, 0)
    row0 = sid * RPT
    for j in range(RPT // CH):
        pltpu.async_copy(rows_v.at[0], acc.at[pl.ds(row0 + j * CH, CH)],
                         sem_s)
    rem = RPT - (RPT // CH) * CH
    pltpu.async_copy(rows_v.at[0].at[pl.ds(0, rem)],
                     acc.at[pl.ds(row0 + RPT - rem, rem)], sem_s)

    @pl.when(sid == 0)
    def _zero_tail():
        pltpu.sync_copy(rows_v.at[0].at[pl.ds(0, N - NS * RPT)],
                        acc.at[pl.ds(NS * RPT, N - NS * RPT)])
    for j in range(RPT // CH):
        pltpu.make_async_copy(rows_v.at[0],
                              acc.at[pl.ds(row0 + j * CH, CH)], sem_s).wait()
    pltpu.make_async_copy(rows_v.at[0].at[pl.ds(0, rem)],
                          acc.at[pl.ds(row0 + RPT - rem, rem)], sem_s).wait()

    _meta_wait(0, 0, 0)
    _build_chunk(0, 0)
    _gather_desc(0, 0).start()
    plsc.subcore_barrier()

    def _cb(i, _):
        e = lax.rem(i, 4)
        m = lax.rem(i, 8)
        q = lax.rem(i, 4)
        mp = lax.rem(i + 6, 8)
        qp = lax.rem(i + 2, 4)

        @pl.when(i >= 4)
        def _drain():
            _scatter_desc(q, lax.rem(i + 4, 8)).wait()
        _meta_wait(i, e, m)
        _build_chunk(e, m)
        _gather_desc(m, q).start()

        @pl.when(i < NCH - 2)
        def _prefetch():
            _meta_start(i + 2, lax.rem(i + 2, 4), lax.rem(i + 2, 8))

        @pl.when(i >= 2)
        def _process():
            _gather_desc(mp, qp).wait()
            _scale(qp, mp)
            _scatter_desc(qp, mp).start(add=True)
        return 0
    lax.fori_loop(1, NCH, _cb, 0)

    for c in (NCH - 2, NCH - 1):
        _gather_desc(c % 8, c % 4).wait()
        _scale(c % 4, c % 8)
        _scatter_desc(c % 4, c % 8).start(add=True)
    for c in (NCH - 4, NCH - 3, NCH - 2, NCH - 1):
        _scatter_desc(c % 4, c % 8).wait()

    plsc.subcore_barrier()
    sl0 = pl.ds(row0, RPT)
    pltpu.sync_copy(acc.at[sl0], out_hbm.at[cid].at[sl0])

    @pl.when(sid == 0)
    def _pub_tail():
        tl = pl.ds(NS * RPT, N - NS * RPT)
        pltpu.sync_copy(acc.at[tl], out_hbm.at[cid].at[tl])


_edge_scatter = functools.partial(
    pl.kernel,
    out_type=jax.ShapeDtypeStruct((NC, N, D), jnp.float32),
    mesh=plsc.VectorSubcoreMesh(core_axis_name="c", subcore_axis_name="s"),
    scratch_types=[
        pltpu.VMEM((4, 3, CH), jnp.int32),
        pltpu.VMEM((8, CH), jnp.int32),
        pltpu.VMEM((8, CH), jnp.int32),
        pltpu.VMEM((8, CH), jnp.float32),
        pltpu.VMEM((4, CH, D), jnp.float32),
        pltpu.VMEM_SHARED((N, D), jnp.float32),
        pltpu.SemaphoreType.DMA,
        pltpu.SemaphoreType.DMA,
        pltpu.SemaphoreType.DMA,
    ],
)(_edge_scatter_body)


def kernel(feat, edge_index, etype, norm, coeff0, bases0, bias0,
           coeff1, bases1, bias1):
    edata = (jnp.stack([edge_index[0], etype, edge_index[1]], 0)
             .reshape(3, NW, NCH, CH).transpose(1, 2, 0, 3))
    norm3 = norm.reshape(NW, NCH, CH)

    w0 = _combine_w(coeff0, bases0)
    w1 = _combine_w(coeff1, bases1)
    t0 = _all_transform0(feat, w0).reshape(R * N, D)
    p0 = _edge_scatter(edata, norm3, t0)
    t1 = _all_transform1(p0, bias0, w1).reshape(R * N, D)
    p1 = _edge_scatter(edata, norm3, t1)
    return _combine_partials(p1, bias1)
